# baseline (speedup 1.0000x reference)
import functools

import jax
import jax.numpy as jnp
from jax import lax
from jax.experimental import pallas as pl
from jax.experimental.pallas import tpu as pltpu
from jax.experimental.pallas import tpu_sc as plsc

_V = 1000
_H = 128
_CH = 80


_SB = 5


def _matmul_body(h_ref, w_ref, o_ref):
    for j in range(_SB):
        o_ref[j] = lax.dot_general(
            w_ref[...], h_ref[j],
            dimension_numbers=(((1,), (1,)), ((), ())),
            preferred_element_type=jnp.float32,
        )


def _matmul_body_alias(h_ref, w_ref, prev_ref, o_ref):
    del prev_ref
    _matmul_body(h_ref, w_ref, o_ref)


def _logits_first(hidden_sb, w, s_part, s_total, b):
    return pl.pallas_call(
        _matmul_body,
        grid=(s_part // _SB,),
        in_specs=[
            pl.BlockSpec((_SB, b, _H), lambda i: (i, 0, 0)),
            pl.BlockSpec((_V, _H), lambda i: (0, 0)),
        ],
        out_specs=pl.BlockSpec((_SB, _V, b), lambda i: (i, 0, 0)),
        out_shape=jax.ShapeDtypeStruct((s_total, _V, b), jnp.float32),
        compiler_params=pltpu.CompilerParams(
            vmem_limit_bytes=100 * 1024 * 1024),
    )(hidden_sb, w)


def _logits_rest(hidden_sb, w, prev, s_off, b):
    s_total = prev.shape[0]
    n = (s_total - s_off) // _SB
    return pl.pallas_call(
        _matmul_body_alias,
        grid=(n,),
        in_specs=[
            pl.BlockSpec((_SB, b, _H), lambda i: (i, 0, 0)),
            pl.BlockSpec((_V, _H), lambda i: (0, 0)),
            pl.BlockSpec(memory_space=pl.ANY),
        ],
        out_specs=pl.BlockSpec(
            (_SB, _V, b), lambda i: (i + s_off // _SB, 0, 0)),
        out_shape=jax.ShapeDtypeStruct((s_total, _V, b), jnp.float32),
        input_output_aliases={2: 0},
        compiler_params=pltpu.CompilerParams(
            vmem_limit_bytes=100 * 1024 * 1024),
    )(hidden_sb, w, prev)


def _make_gather(total_rows):
    info = plsc.get_sparse_core_info()
    nc, ns = info.num_cores, info.num_subcores
    nw = nc * ns
    assert total_rows % (nw * _CH) == 0
    rows_per_w = total_rows // nw
    n_chunks = rows_per_w // _CH
    assert n_chunks >= 2 and n_chunks % 2 == 0
    mesh = plsc.VectorSubcoreMesh(core_axis_name="c", subcore_axis_name="s")

    @functools.partial(
        pl.kernel,
        mesh=mesh,
        out_type=jax.ShapeDtypeStruct((total_rows, _H), jnp.float32),
        scratch_types=[
            pltpu.VMEM_SHARED((_V, _H), jnp.float32),
            pltpu.VMEM((rows_per_w,), jnp.int32),
            pltpu.VMEM((_CH, _H), jnp.float32),
            pltpu.VMEM((_CH, _H), jnp.float32),
            pltpu.SemaphoreType.DMA,
            pltpu.SemaphoreType.DMA,
            pltpu.SemaphoreType.DMA,
            pltpu.SemaphoreType.DMA,
        ],
    )
    def gather_k(table_hbm, idx_hbm, out_hbm, tab_sh, idx_v,
                 buf0, buf1, g0, g1, s0, s1):
        cid = lax.axis_index("c")
        sid = lax.axis_index("s")
        wid = sid * nc + cid
        base = wid * rows_per_w

        @pl.when(sid == 0)
        def _stage():
            pltpu.sync_copy(table_hbm, tab_sh)

        pltpu.sync_copy(idx_hbm.at[pl.ds(base, rows_per_w)], idx_v)
        plsc.subcore_barrier()

        bufs = (buf0, buf1)
        gsem = (g0, g1)
        ssem = (s0, s1)

        def g_copy(i, b):
            off = pl.multiple_of(i * _CH, 8)
            return pltpu.make_async_copy(
                tab_sh.at[idx_v.at[pl.ds(off, _CH)]], bufs[b], gsem[b])

        def s_copy(i, b):
            return pltpu.make_async_copy(
                bufs[b], out_hbm.at[pl.ds(base + i * _CH, _CH)], ssem[b])

        g_copy(0, 0).start()
        g_copy(1, 1).start()

        def outer(io, carry):
            for b in range(2):
                i = io * 2 + b
                g_copy(i, b).wait()
                s_copy(i, b).start()
                nxt = i + 2

                @pl.when(nxt < n_chunks)
                def _refill():
                    s_copy(i, b).wait()
                    g_copy(nxt, b).start()

            return carry

        lax.fori_loop(0, n_chunks // 2, outer, 0)
        s_copy(n_chunks - 2, 0).wait()
        s_copy(n_chunks - 1, 1).wait()

    return gather_k


_S1 = 30


def kernel(input_ids, embed_weight, linear_weight):
    b, s = input_ids.shape
    ids_sb = input_ids.T.reshape(b * s).astype(jnp.int32)
    n1 = _S1 * b
    hidden1 = _make_gather(n1)(embed_weight, ids_sb[:n1])
    hidden2 = _make_gather(b * s - n1)(embed_weight, ids_sb[n1:])
    part = _logits_first(hidden1.reshape(_S1, b, _H), linear_weight, _S1, s, b)
    logits_svb = _logits_rest(
        hidden2.reshape(s - _S1, b, _H), linear_weight, part, _S1, b)
    return jnp.transpose(logits_svb, (2, 0, 1))


# speedup vs baseline: 1.0368x; 1.0368x over previous
import functools

import jax
import jax.numpy as jnp
from jax import lax
from jax.experimental import pallas as pl
from jax.experimental.pallas import tpu as pltpu
from jax.experimental.pallas import tpu_sc as plsc

_V = 1000
_H = 128
_CH = 80
_SB = 5


def _matmul_body(h_ref, w_ref, o_ref):
    for j in range(_SB):
        o_ref[j] = lax.dot_general(
            w_ref[...], h_ref[j],
            dimension_numbers=(((1,), (1,)), ((), ())),
            preferred_element_type=jnp.float32,
        )


def _logits_svb(hidden_sb, w, s, b):
    return pl.pallas_call(
        _matmul_body,
        grid=(s // _SB,),
        in_specs=[
            pl.BlockSpec((_SB, b, _H), lambda i: (i, 0, 0)),
            pl.BlockSpec((_V, _H), lambda i: (0, 0)),
        ],
        out_specs=pl.BlockSpec((_SB, _V, b), lambda i: (i, 0, 0)),
        out_shape=jax.ShapeDtypeStruct((s, _V, b), jnp.float32),
        compiler_params=pltpu.CompilerParams(
            vmem_limit_bytes=100 * 1024 * 1024),
    )(hidden_sb, w)


def _make_gather(total_rows):
    info = plsc.get_sparse_core_info()
    nc, ns = info.num_cores, info.num_subcores
    nw = nc * ns
    assert total_rows % (nw * _CH) == 0
    rows_per_w = total_rows // nw
    n_chunks = rows_per_w // _CH
    nbuf = 4
    assert n_chunks >= nbuf and n_chunks % nbuf == 0
    mesh = plsc.VectorSubcoreMesh(core_axis_name="c", subcore_axis_name="s")

    @functools.partial(
        pl.kernel,
        mesh=mesh,
        out_type=jax.ShapeDtypeStruct((total_rows, _H), jnp.float32),
        scratch_types=[
            pltpu.VMEM_SHARED((_V, _H), jnp.float32),
            pltpu.VMEM((rows_per_w,), jnp.int32),
        ] + [pltpu.VMEM((_CH, _H), jnp.float32)] * 4
          + [pltpu.SemaphoreType.DMA] * 8,
    )
    def gather_k(table_hbm, idx_hbm, out_hbm, tab_sh, idx_v,
                 buf0, buf1, buf2, buf3,
                 g0, g1, g2, g3, s0, s1, s2, s3):
        cid = lax.axis_index("c")
        sid = lax.axis_index("s")
        wid = sid * nc + cid
        base = wid * rows_per_w

        @pl.when(sid == 0)
        def _stage():
            pltpu.sync_copy(table_hbm, tab_sh)

        pltpu.sync_copy(idx_hbm.at[pl.ds(base, rows_per_w)], idx_v)
        plsc.subcore_barrier()

        bufs = (buf0, buf1, buf2, buf3)
        gsem = (g0, g1, g2, g3)
        ssem = (s0, s1, s2, s3)

        def g_copy(i, b):
            off = pl.multiple_of(i * _CH, 8)
            return pltpu.make_async_copy(
                tab_sh.at[idx_v.at[pl.ds(off, _CH)]], bufs[b], gsem[b])

        def s_copy(i, b):
            return pltpu.make_async_copy(
                bufs[b], out_hbm.at[pl.ds(base + i * _CH, _CH)], ssem[b])

        for b in range(nbuf):
            g_copy(b, b).start()

        def outer(io, carry):
            for b in range(nbuf):
                i = io * nbuf + b
                g_copy(i, b).wait()
                s_copy(i, b).start()
                nxt = i + nbuf

                @pl.when(nxt < n_chunks)
                def _refill():
                    s_copy(i, b).wait()
                    g_copy(nxt, b).start()

            return carry

        lax.fori_loop(0, n_chunks // nbuf, outer, 0)
        for b in range(nbuf):
            s_copy(n_chunks - nbuf + b, b).wait()

    return gather_k


def kernel(input_ids, embed_weight, linear_weight):
    b, s = input_ids.shape
    ids_sb = input_ids.T.reshape(b * s).astype(jnp.int32)
    hidden = _make_gather(b * s)(embed_weight, ids_sb)
    logits_svb = _logits_svb(hidden.reshape(s, b, _H), linear_weight, s, b)
    return jnp.transpose(logits_svb, (2, 0, 1))
